# baseline (speedup 1.0000x reference)
import functools
import math

import jax
import jax.numpy as jnp
from jax import lax
from jax.experimental import pallas as pl
from jax.experimental.pallas import tpu as pltpu
from jax.experimental.pallas import tpu_sc as plsc

D_MODEL = 64
SCALE = math.sqrt(D_MODEL)
CHUNK = 128


@functools.lru_cache(maxsize=None)
def _build(B, V, n_rows, n_cols):
    info = plsc.get_sparse_core_info()
    NW = info.num_cores * info.num_subcores
    NC = info.num_cores
    assert B % (NW * CHUNK) == 0
    b_per_w = B // NW
    n_chunks = b_per_w // CHUNK

    mesh = plsc.VectorSubcoreMesh(core_axis_name="c", subcore_axis_name="s")

    @functools.partial(
        pl.kernel,
        mesh=mesh,
        compiler_params=pltpu.CompilerParams(use_tc_tiling_on_sc=False),
        out_type=jax.ShapeDtypeStruct((B, D_MODEL), jnp.float32),
        scratch_types=[
            pltpu.VMEM((n_chunks, CHUNK), jnp.int32),
            pltpu.VMEM((CHUNK, D_MODEL), jnp.float32),
            pltpu.SemaphoreType.DMA,
        ],
    )
    def emb_kernel(idx_hbm, table_hbm, out_hbm, idx_v, rows_v, sem):
        wid = lax.axis_index("s") * NC + lax.axis_index("c")
        base = wid * b_per_w
        pltpu.sync_copy(idx_hbm.at[wid], idx_v)

        def chunk_body(j, _):
            pltpu.async_copy(table_hbm.at[idx_v.at[j]], rows_v, sem).wait()

            def row_body(r, _):
                for c in range(D_MODEL // 16):
                    sl = (r, pl.ds(c * 16, 16))
                    rows_v[sl] = rows_v[sl] * SCALE
                return 0

            lax.fori_loop(0, CHUNK, row_body, 0, unroll=4)
            pltpu.sync_copy(rows_v, out_hbm.at[pl.ds(base + j * CHUNK, CHUNK)])
            return 0

        lax.fori_loop(0, n_chunks, chunk_body, 0)

    def run(x, table):
        idx3d = x.reshape(-1).astype(jnp.int32).reshape(NW, n_chunks, CHUNK)
        out = emb_kernel(idx3d, table)
        return out.reshape(n_rows, n_cols, D_MODEL)

    return run


def kernel(x, table):
    n_rows, n_cols = x.shape
    V = table.shape[0]
    return _build(n_rows * n_cols, V, n_rows, n_cols)(x, table)


# speedup vs baseline: 1.0647x; 1.0647x over previous
import functools
import math

import jax
import jax.numpy as jnp
from jax import lax
from jax.experimental import pallas as pl
from jax.experimental.pallas import tpu as pltpu
from jax.experimental.pallas import tpu_sc as plsc

D_MODEL = 64
SCALE = math.sqrt(D_MODEL)
CHUNK = 128
NBUF = 5


@functools.lru_cache(maxsize=None)
def _build(B, V, n_rows, n_cols):
    info = plsc.get_sparse_core_info()
    NW = info.num_cores * info.num_subcores
    NC = info.num_cores
    assert B % (NW * CHUNK) == 0
    b_per_w = B // NW
    n_chunks = b_per_w // CHUNK
    assert n_chunks % NBUF == 0

    mesh = plsc.VectorSubcoreMesh(core_axis_name="c", subcore_axis_name="s")

    scratch = [pltpu.VMEM((n_chunks, CHUNK), jnp.int32)]
    scratch += [pltpu.VMEM((CHUNK, D_MODEL), jnp.float32) for _ in range(NBUF)]
    scratch += [pltpu.SemaphoreType.DMA for _ in range(2 * NBUF)]

    @functools.partial(
        pl.kernel,
        mesh=mesh,
        compiler_params=pltpu.CompilerParams(use_tc_tiling_on_sc=False),
        out_type=jax.ShapeDtypeStruct((B, D_MODEL), jnp.float32),
        scratch_types=scratch,
    )
    def emb_kernel(idx_hbm, table_hbm, out_hbm, idx_v, *bufs_and_sems):
        bufs = bufs_and_sems[:NBUF]
        sem_g = bufs_and_sems[NBUF:2 * NBUF]
        sem_s = bufs_and_sems[2 * NBUF:]
        wid = lax.axis_index("s") * NC + lax.axis_index("c")
        base = wid * b_per_w

        pltpu.sync_copy(idx_hbm.at[wid], idx_v)

        def gather_start(c, b):
            pltpu.make_async_copy(
                table_hbm.at[idx_v.at[c]], bufs[b], sem_g[b]).start()

        def gather_wait(b):
            pltpu.make_async_copy(
                table_hbm.at[idx_v.at[0]], bufs[b], sem_g[b]).wait()

        def store_start(c, b):
            pltpu.make_async_copy(
                bufs[b], out_hbm.at[pl.ds(base + c * CHUNK, CHUNK)],
                sem_s[b]).start()

        def store_wait(b):
            pltpu.make_async_copy(
                bufs[b], out_hbm.at[pl.ds(base, CHUNK)], sem_s[b]).wait()

        for b in range(NBUF - 1):
            gather_start(b, b)

        def outer_body(o, _):
            for b in range(NBUF):
                c = o + b
                gather_wait(b)

                @plsc.parallel_loop(0, CHUNK, step=1, unroll=4)
                def scale_row(r):
                    for k in range(D_MODEL // 16):
                        sl = (r, pl.ds(k * 16, 16))
                        bufs[b][sl] = bufs[b][sl] * SCALE

                store_start(c, b)
                bp = (b - 1) % NBUF
                p = c + NBUF - 1

                @pl.when((c >= 1) & (p < n_chunks))
                def _():
                    store_wait(bp)

                @pl.when(p < n_chunks)
                def _():
                    gather_start(p, bp)

            return 0

        lax.fori_loop(0, n_chunks // NBUF, lambda i, _: outer_body(i * NBUF, 0), 0)

        for b in range(NBUF):
            store_wait(b)

    def run(x, table):
        idx3d = x.reshape(-1).astype(jnp.int32).reshape(NW, n_chunks, CHUNK)
        out = emb_kernel(idx3d, table)
        return out.reshape(n_rows, n_cols, D_MODEL)

    return run


def kernel(x, table):
    n_rows, n_cols = x.shape
    V = table.shape[0]
    return _build(n_rows * n_cols, V, n_rows, n_cols)(x, table)
